# baseline (speedup 1.0000x reference)
import functools

import jax
import jax.numpy as jnp
from jax import lax
from jax.experimental import pallas as pl
from jax.experimental.pallas import tpu as pltpu
from jax.experimental.pallas import tpu_sc as plsc

D_MODEL = 768
N_HEADS = 12
HG = N_HEADS // 2
DH = D_MODEL // N_HEADS
ROW = 2 * DH
BS = 16
LT = 432
LTA = 144
NW = 32
SCCH = 128


def _qkvpack_body(x_ref, qk_ref, wv_ref, bv_ref, qkva_ref, qkvb_ref):
    xt = x_ref[0]
    dn = (((1,), (1,)), ((), ()))
    v = lax.dot_general(xt, wv_ref[...], dn,
                        preferred_element_type=jnp.float32) + bv_ref[0]
    qk = qk_ref[0]
    for h in range(N_HEADS):
        dst = qkva_ref if h < HG else qkvb_ref
        dst[h % HG, 0] = jnp.concatenate(
            [qk[:, h * DH:(h + 1) * DH], v[:, h * DH:(h + 1) * DH]], axis=1)


def _attn_body(qkv_ref, o_ref):
    q = qkv_ref[0, :, :DH]
    v = qkv_ref[0, :, DH:]
    s = lax.dot_general(q, q, (((1,), (1,)), ((), ())),
                        preferred_element_type=jnp.float32)
    r = lax.broadcasted_iota(jnp.int32, (LTA, LTA), 0)
    c = lax.broadcasted_iota(jnp.int32, (LTA, LTA), 1)
    mask = ((r // BS) == (c // BS)) & (r != c)
    s = jnp.where(mask, s, -1e30)
    m = jnp.max(s, axis=1, keepdims=True)
    p = jnp.exp(s - m)
    denom = jnp.sum(p, axis=1, keepdims=True)
    o = lax.dot_general(p, v, (((1,), (0,)), ((), ())),
                        preferred_element_type=jnp.float32)
    o_ref[0] = jnp.concatenate([o / denom, jnp.zeros_like(o)], axis=1)


def _outproj_body(oa_ref, ob_ref, wo_ref, bo_ref, out_ref):
    acc = jnp.zeros((LT, D_MODEL), jnp.float32) + bo_ref[0]
    for h in range(N_HEADS):
        src = oa_ref if h < HG else ob_ref
        acc = acc + lax.dot_general(
            src[h % HG, 0, :, :DH], wo_ref[:, h * DH:(h + 1) * DH],
            (((1,), (1,)), ((), ())), preferred_element_type=jnp.float32)
    out_ref[0] = acc


def _make_sc_permute(G, Lp, reverse):
    HALF = Lp // 2
    NCH = HALF // SCCH
    TAIL = HALF - NCH * SCCH
    NU = 2 * G
    TRIPS = (NU + NW - 1) // NW
    mesh = plsc.VectorSubcoreMesh(core_axis_name="c", subcore_axis_name="s")

    def body(src, idxp, dst, idx_v, idx_t, rows, rows_t, sem):
        wid = lax.axis_index("s") * 2 + lax.axis_index("c")

        def do_chunk(gbase, off, n, idx_ref, rows_ref):
            pltpu.sync_copy(idxp.at[pl.ds(gbase + off, n)], idx_ref)
            if reverse:
                pltpu.async_copy(src.at[idx_ref], rows_ref, sem).wait()
                pltpu.sync_copy(rows_ref, dst.at[pl.ds(gbase + off, n)])
            else:
                pltpu.sync_copy(src.at[pl.ds(gbase + off, n)], rows_ref)
                pltpu.async_copy(rows_ref, dst.at[idx_ref], sem).wait()

        def unit_body(u, carry):
            unit = u * NW + wid

            @pl.when(unit < NU)
            def _():
                gbase = (unit // 2) * Lp + (unit % 2) * HALF

                def chunk_body(j, carry2):
                    do_chunk(gbase, j * SCCH, SCCH, idx_v, rows)
                    return carry2

                lax.fori_loop(0, NCH, chunk_body, 0)
                do_chunk(gbase, NCH * SCCH, TAIL, idx_t, rows_t)

            return carry

        lax.fori_loop(0, TRIPS, unit_body, 0)

    return functools.partial(
        pl.kernel, body, mesh=mesh,
        out_type=jax.ShapeDtypeStruct((G * Lp, ROW), jnp.float32),
        scratch_types=[
            pltpu.VMEM((SCCH,), jnp.int32),
            pltpu.VMEM((TAIL,), jnp.int32),
            pltpu.VMEM((SCCH, ROW), jnp.float32),
            pltpu.VMEM((TAIL, ROW), jnp.float32),
            pltpu.SemaphoreType.DMA,
        ])()


SORT_N = 16384
SORT_C = 128


def _bitonic_body(a_ref, idx_ref):
    Rr = SORT_N // SORT_C
    key = a_ref[0]
    r = lax.broadcasted_iota(jnp.int32, (Rr, SORT_C), 0)
    c = lax.broadcasted_iota(jnp.int32, (Rr, SORT_C), 1)
    ilin = r * SORT_C + c
    idx = ilin
    k = 2
    while k <= SORT_N:
        j = k // 2
        while j >= 1:
            if j >= SORT_C:
                s, axis, bit_src = j // SORT_C, 0, r
            else:
                s, axis, bit_src = j, 1, c
            bit = (bit_src & s) != 0
            pk = jnp.where(bit, jnp.roll(key, s, axis), jnp.roll(key, -s, axis))
            pi = jnp.where(bit, jnp.roll(idx, s, axis), jnp.roll(idx, -s, axis))
            lt = (key < pk) | ((key == pk) & (idx < pi))
            desc = (ilin & k) != 0
            take = jnp.logical_xor(jnp.logical_xor(lt, bit), desc)
            key = jnp.where(take, key, pk)
            idx = jnp.where(take, idx, pi)
            j //= 2
        k *= 2
    idx_ref[0] = idx


def _bitonic_argsort(angles2d):
    G, Lp = angles2d.shape
    pad = jnp.full((G, SORT_N - Lp), jnp.inf, jnp.float32)
    ap = jnp.concatenate([angles2d, pad], axis=1).reshape(G, SORT_N // SORT_C, SORT_C)
    idx = pl.pallas_call(
        _bitonic_body,
        grid=(G,),
        in_specs=[pl.BlockSpec((1, SORT_N // SORT_C, SORT_C), lambda g: (g, 0, 0))],
        out_specs=pl.BlockSpec((1, SORT_N // SORT_C, SORT_C), lambda g: (g, 0, 0)),
        out_shape=jax.ShapeDtypeStruct((G, SORT_N // SORT_C, SORT_C), jnp.int32),
    )(ap)
    return idx.reshape(G, SORT_N)[:, :Lp]


def kernel(x, W_qk, b_qk, W_v, b_v, W_o, b_o):
    Bn, Ln, D = x.shape
    H, dh, bs = N_HEADS, DH, BS
    pad_len = bs - (Ln % bs)
    xp = jnp.concatenate([x, jnp.zeros((Bn, pad_len, D), x.dtype)], axis=1)
    Lp = xp.shape[1]
    nt = Lp // LT
    GB = HG * Bn

    qk_all = xp @ W_qk.T + b_qk
    angles_l = []
    for h in range(H):
        qk = qk_all[:, :, h * dh:(h + 1) * dh]
        R = jax.random.normal(jax.random.fold_in(jax.random.key(42), h),
                              (dh, 2), dtype=jnp.float32)
        hout = lax.stop_gradient(qk) @ R
        angles_l.append(jnp.arctan(hout[:, :, 0] / hout[:, :, 1]))
    angles = jnp.stack(angles_l, axis=0)

    qkv_a, qkv_b = pl.pallas_call(
        _qkvpack_body,
        grid=(Bn, nt),
        in_specs=[
            pl.BlockSpec((1, LT, D), lambda b, t: (b, t, 0)),
            pl.BlockSpec((1, LT, D), lambda b, t: (b, t, 0)),
            pl.BlockSpec((D, D), lambda b, t: (0, 0)),
            pl.BlockSpec((1, D), lambda b, t: (0, 0)),
        ],
        out_specs=[
            pl.BlockSpec((HG, 1, LT, ROW), lambda b, t: (0, b, t, 0)),
            pl.BlockSpec((HG, 1, LT, ROW), lambda b, t: (0, b, t, 0)),
        ],
        out_shape=[
            jax.ShapeDtypeStruct((HG, Bn, Lp, ROW), jnp.float32),
            jax.ShapeDtypeStruct((HG, Bn, Lp, ROW), jnp.float32),
        ],
    )(xp, qk_all, W_v, b_v.reshape(1, D))

    offs = (jnp.arange(GB, dtype=jnp.int32) * Lp)[:, None]
    o_groups = []
    for g, qkv_g in ((0, qkv_a), (1, qkv_b)):
        ang_g = lax.dynamic_slice_in_dim(angles, g * HG, HG, axis=0)
        idx_g = _bitonic_argsort(ang_g.reshape(GB, Lp))
        idxp_g = (idx_g + offs).reshape(GB * Lp).astype(jnp.int32)
        qkv_s = _make_sc_permute(GB, Lp, reverse=False)(
            qkv_g.reshape(GB * Lp, ROW), idxp_g)
        o_s = pl.pallas_call(
            _attn_body,
            grid=(GB, Lp // LTA),
            in_specs=[pl.BlockSpec((1, LTA, ROW), lambda gg, t: (gg, t, 0))],
            out_specs=pl.BlockSpec((1, LTA, ROW), lambda gg, t: (gg, t, 0)),
            out_shape=jax.ShapeDtypeStruct((GB, Lp, ROW), jnp.float32),
        )(qkv_s.reshape(GB, Lp, ROW))
        o_g = _make_sc_permute(GB, Lp, reverse=True)(
            o_s.reshape(GB * Lp, ROW), idxp_g)
        o_groups.append(o_g.reshape(HG, Bn, Lp, ROW))

    out = pl.pallas_call(
        _outproj_body,
        grid=(Bn, nt),
        in_specs=[
            pl.BlockSpec((HG, 1, LT, ROW), lambda b, t: (0, b, t, 0)),
            pl.BlockSpec((HG, 1, LT, ROW), lambda b, t: (0, b, t, 0)),
            pl.BlockSpec((D, D), lambda b, t: (0, 0)),
            pl.BlockSpec((1, D), lambda b, t: (0, 0)),
        ],
        out_specs=pl.BlockSpec((1, LT, D), lambda b, t: (b, t, 0)),
        out_shape=jax.ShapeDtypeStruct((Bn, Lp, D), jnp.float32),
    )(o_groups[0], o_groups[1], W_o, b_o.reshape(1, D))
    return out[:, :Ln]


# speedup vs baseline: 1.4103x; 1.4103x over previous
import functools

import jax
import jax.numpy as jnp
from jax import lax
from jax.experimental import pallas as pl
from jax.experimental.pallas import tpu as pltpu
from jax.experimental.pallas import tpu_sc as plsc

D_MODEL = 768
N_HEADS = 12
HG = N_HEADS // 2
DH = D_MODEL // N_HEADS
ROW = 2 * DH
BS = 16
LT = 432
LTA = 432
NW = 32
SCCH = 128


def _qkvpack_body(x_ref, qk_ref, wv_ref, bv_ref, qkva_ref, qkvb_ref):
    xt = x_ref[0]
    dn = (((1,), (1,)), ((), ()))
    v = lax.dot_general(xt, wv_ref[...], dn,
                        preferred_element_type=jnp.float32) + bv_ref[0]
    qk = qk_ref[0]
    for h in range(N_HEADS):
        dst = qkva_ref if h < HG else qkvb_ref
        dst[h % HG, 0] = jnp.concatenate(
            [qk[:, h * DH:(h + 1) * DH], v[:, h * DH:(h + 1) * DH]], axis=1)


def _attn_body(qkv_ref, o_ref):
    q = qkv_ref[0, :, :DH]
    v = qkv_ref[0, :, DH:]
    s = lax.dot_general(q, q, (((1,), (1,)), ((), ())),
                        preferred_element_type=jnp.float32)
    r = lax.broadcasted_iota(jnp.int32, (LTA, LTA), 0)
    c = lax.broadcasted_iota(jnp.int32, (LTA, LTA), 1)
    mask = ((r // BS) == (c // BS)) & (r != c)
    s = jnp.where(mask, s, -1e30)
    m = jnp.max(s, axis=1, keepdims=True)
    p = jnp.exp(s - m)
    denom = jnp.sum(p, axis=1, keepdims=True)
    o = lax.dot_general(p, v, (((1,), (0,)), ((), ())),
                        preferred_element_type=jnp.float32)
    o_ref[0] = jnp.concatenate([o / denom, jnp.zeros_like(o)], axis=1)


def _outproj_body(oa_ref, ob_ref, wo_ref, bo_ref, out_ref):
    acc = jnp.zeros((LT, D_MODEL), jnp.float32) + bo_ref[0]
    for h in range(N_HEADS):
        src = oa_ref if h < HG else ob_ref
        acc = acc + lax.dot_general(
            src[h % HG, 0, :, :DH], wo_ref[:, h * DH:(h + 1) * DH],
            (((1,), (1,)), ((), ())), preferred_element_type=jnp.float32)
    out_ref[0] = acc


def _make_sc_permute(G, Lp, reverse):
    HALF = Lp // 2
    NCH = HALF // SCCH
    TAIL = HALF - NCH * SCCH
    NU = 2 * G
    TRIPS = (NU + NW - 1) // NW
    mesh = plsc.VectorSubcoreMesh(core_axis_name="c", subcore_axis_name="s")

    def body(src, idxp, dst, idx_v, idx_t, rows, rows_t, sem):
        wid = lax.axis_index("s") * 2 + lax.axis_index("c")

        def do_chunk(gbase, off, n, idx_ref, rows_ref):
            pltpu.sync_copy(idxp.at[pl.ds(gbase + off, n)], idx_ref)
            if reverse:
                pltpu.async_copy(src.at[idx_ref], rows_ref, sem).wait()
                pltpu.sync_copy(rows_ref, dst.at[pl.ds(gbase + off, n)])
            else:
                pltpu.sync_copy(src.at[pl.ds(gbase + off, n)], rows_ref)
                pltpu.async_copy(rows_ref, dst.at[idx_ref], sem).wait()

        def unit_body(u, carry):
            unit = u * NW + wid

            @pl.when(unit < NU)
            def _():
                gbase = (unit // 2) * Lp + (unit % 2) * HALF

                def chunk_body(j, carry2):
                    do_chunk(gbase, j * SCCH, SCCH, idx_v, rows)
                    return carry2

                lax.fori_loop(0, NCH, chunk_body, 0)
                do_chunk(gbase, NCH * SCCH, TAIL, idx_t, rows_t)

            return carry

        lax.fori_loop(0, TRIPS, unit_body, 0)

    return functools.partial(
        pl.kernel, body, mesh=mesh,
        out_type=jax.ShapeDtypeStruct((G * Lp, ROW), jnp.float32),
        scratch_types=[
            pltpu.VMEM((SCCH,), jnp.int32),
            pltpu.VMEM((TAIL,), jnp.int32),
            pltpu.VMEM((SCCH, ROW), jnp.float32),
            pltpu.VMEM((TAIL, ROW), jnp.float32),
            pltpu.SemaphoreType.DMA,
        ])()


SORT_N = 16384
SORT_C = 128


def _bitonic_body(a_ref, idx_ref):
    Rr = SORT_N // SORT_C
    key = a_ref[0]
    r = lax.broadcasted_iota(jnp.int32, (Rr, SORT_C), 0)
    c = lax.broadcasted_iota(jnp.int32, (Rr, SORT_C), 1)
    ilin = r * SORT_C + c
    idx = ilin
    k = 2
    while k <= SORT_N:
        j = k // 2
        while j >= 1:
            if j >= SORT_C:
                s, axis, bit_src = j // SORT_C, 0, r
            else:
                s, axis, bit_src = j, 1, c
            bit = (bit_src & s) != 0
            pk = jnp.where(bit, jnp.roll(key, s, axis), jnp.roll(key, -s, axis))
            pi = jnp.where(bit, jnp.roll(idx, s, axis), jnp.roll(idx, -s, axis))
            lt = (key < pk) | ((key == pk) & (idx < pi))
            desc = (ilin & k) != 0
            take = jnp.logical_xor(jnp.logical_xor(lt, bit), desc)
            key = jnp.where(take, key, pk)
            idx = jnp.where(take, idx, pi)
            j //= 2
        k *= 2
    idx_ref[0] = idx


def _bitonic_argsort(angles2d):
    G, Lp = angles2d.shape
    pad = jnp.full((G, SORT_N - Lp), jnp.inf, jnp.float32)
    ap = jnp.concatenate([angles2d, pad], axis=1).reshape(G, SORT_N // SORT_C, SORT_C)
    idx = pl.pallas_call(
        _bitonic_body,
        grid=(G,),
        in_specs=[pl.BlockSpec((1, SORT_N // SORT_C, SORT_C), lambda g: (g, 0, 0))],
        out_specs=pl.BlockSpec((1, SORT_N // SORT_C, SORT_C), lambda g: (g, 0, 0)),
        out_shape=jax.ShapeDtypeStruct((G, SORT_N // SORT_C, SORT_C), jnp.int32),
    )(ap)
    return idx.reshape(G, SORT_N)[:, :Lp]


def kernel(x, W_qk, b_qk, W_v, b_v, W_o, b_o):
    Bn, Ln, D = x.shape
    H, dh, bs = N_HEADS, DH, BS
    pad_len = bs - (Ln % bs)
    xp = jnp.concatenate([x, jnp.zeros((Bn, pad_len, D), x.dtype)], axis=1)
    Lp = xp.shape[1]
    nt = Lp // LT
    GB = HG * Bn

    qk_all = xp @ W_qk.T + b_qk
    angles_l = []
    for h in range(H):
        qk = qk_all[:, :, h * dh:(h + 1) * dh]
        R = jax.random.normal(jax.random.fold_in(jax.random.key(42), h),
                              (dh, 2), dtype=jnp.float32)
        hout = lax.stop_gradient(qk) @ R
        angles_l.append(jnp.arctan(hout[:, :, 0] / hout[:, :, 1]))
    angles = jnp.stack(angles_l, axis=0)

    qkv_a, qkv_b = pl.pallas_call(
        _qkvpack_body,
        grid=(Bn, nt),
        in_specs=[
            pl.BlockSpec((1, LT, D), lambda b, t: (b, t, 0)),
            pl.BlockSpec((1, LT, D), lambda b, t: (b, t, 0)),
            pl.BlockSpec((D, D), lambda b, t: (0, 0)),
            pl.BlockSpec((1, D), lambda b, t: (0, 0)),
        ],
        out_specs=[
            pl.BlockSpec((HG, 1, LT, ROW), lambda b, t: (0, b, t, 0)),
            pl.BlockSpec((HG, 1, LT, ROW), lambda b, t: (0, b, t, 0)),
        ],
        out_shape=[
            jax.ShapeDtypeStruct((HG, Bn, Lp, ROW), jnp.float32),
            jax.ShapeDtypeStruct((HG, Bn, Lp, ROW), jnp.float32),
        ],
    )(xp, qk_all, W_v, b_v.reshape(1, D))

    offs = (jnp.arange(GB, dtype=jnp.int32) * Lp)[:, None]
    o_groups = []
    for g, qkv_g in ((0, qkv_a), (1, qkv_b)):
        ang_g = lax.dynamic_slice_in_dim(angles, g * HG, HG, axis=0)
        idx_g = _bitonic_argsort(ang_g.reshape(GB, Lp))
        idxp_g = (idx_g + offs).reshape(GB * Lp).astype(jnp.int32)
        qkv_s = _make_sc_permute(GB, Lp, reverse=False)(
            qkv_g.reshape(GB * Lp, ROW), idxp_g)
        o_s = pl.pallas_call(
            _attn_body,
            grid=(GB, Lp // LTA),
            in_specs=[pl.BlockSpec((1, LTA, ROW), lambda gg, t: (gg, t, 0))],
            out_specs=pl.BlockSpec((1, LTA, ROW), lambda gg, t: (gg, t, 0)),
            out_shape=jax.ShapeDtypeStruct((GB, Lp, ROW), jnp.float32),
        )(qkv_s.reshape(GB, Lp, ROW))
        o_g = _make_sc_permute(GB, Lp, reverse=True)(
            o_s.reshape(GB * Lp, ROW), idxp_g)
        o_groups.append(o_g.reshape(HG, Bn, Lp, ROW))

    out = pl.pallas_call(
        _outproj_body,
        grid=(Bn, nt),
        in_specs=[
            pl.BlockSpec((HG, 1, LT, ROW), lambda b, t: (0, b, t, 0)),
            pl.BlockSpec((HG, 1, LT, ROW), lambda b, t: (0, b, t, 0)),
            pl.BlockSpec((D, D), lambda b, t: (0, 0)),
            pl.BlockSpec((1, D), lambda b, t: (0, 0)),
        ],
        out_specs=pl.BlockSpec((1, LT, D), lambda b, t: (b, t, 0)),
        out_shape=jax.ShapeDtypeStruct((Bn, Lp, D), jnp.float32),
    )(o_groups[0], o_groups[1], W_o, b_o.reshape(1, D))
    return out[:, :Ln]


# speedup vs baseline: 1.5050x; 1.0672x over previous
import functools

import jax
import jax.numpy as jnp
from jax import lax
from jax.experimental import pallas as pl
from jax.experimental.pallas import tpu as pltpu
from jax.experimental.pallas import tpu_sc as plsc

D_MODEL = 768
N_HEADS = 12
NGRP = 3
HG = N_HEADS // NGRP
DH = D_MODEL // N_HEADS
ROW = 2 * DH
BS = 16
LT = 432
LTA = 432
NW = 32
SCCH = 128


def _qkvpack_body(x_ref, qk_ref, wv_ref, bv_ref, *qkv_refs):
    xt = x_ref[0]
    dn = (((1,), (1,)), ((), ()))
    v = lax.dot_general(xt, wv_ref[...], dn,
                        preferred_element_type=jnp.float32) + bv_ref[0]
    qk = qk_ref[0]
    for h in range(N_HEADS):
        qkv_refs[h // HG][h % HG, 0] = jnp.concatenate(
            [qk[:, h * DH:(h + 1) * DH], v[:, h * DH:(h + 1) * DH]], axis=1)


def _attn_body(qkv_ref, o_ref):
    q = qkv_ref[0, :, :DH]
    v = qkv_ref[0, :, DH:]
    s = lax.dot_general(q, q, (((1,), (1,)), ((), ())),
                        preferred_element_type=jnp.float32)
    r = lax.broadcasted_iota(jnp.int32, (LTA, LTA), 0)
    c = lax.broadcasted_iota(jnp.int32, (LTA, LTA), 1)
    mask = ((r // BS) == (c // BS)) & (r != c)
    s = jnp.where(mask, s, -1e30)
    m = jnp.max(s, axis=1, keepdims=True)
    p = jnp.exp(s - m)
    denom = jnp.sum(p, axis=1, keepdims=True)
    o = lax.dot_general(p, v, (((1,), (0,)), ((), ())),
                        preferred_element_type=jnp.float32)
    o_ref[0] = jnp.concatenate([o / denom, jnp.zeros_like(o)], axis=1)


def _outproj_body(*refs):
    o_refs, (wo_ref, bo_ref, out_ref) = refs[:NGRP], refs[NGRP:]
    acc = jnp.zeros((LT, D_MODEL), jnp.float32) + bo_ref[0]
    for h in range(N_HEADS):
        src = o_refs[h // HG]
        acc = acc + lax.dot_general(
            src[h % HG, 0, :, :DH], wo_ref[:, h * DH:(h + 1) * DH],
            (((1,), (1,)), ((), ())), preferred_element_type=jnp.float32)
    out_ref[0] = acc


def _make_sc_permute(G, Lp, reverse):
    HALF = Lp // 2
    NCH = HALF // SCCH
    TAIL = HALF - NCH * SCCH
    NU = 2 * G
    TRIPS = (NU + NW - 1) // NW
    mesh = plsc.VectorSubcoreMesh(core_axis_name="c", subcore_axis_name="s")

    def body(src, idxp, dst, idx_v, idx_t, rows, rows_t, sem):
        wid = lax.axis_index("s") * 2 + lax.axis_index("c")

        def do_chunk(gbase, off, n, idx_ref, rows_ref):
            pltpu.sync_copy(idxp.at[pl.ds(gbase + off, n)], idx_ref)
            if reverse:
                pltpu.async_copy(src.at[idx_ref], rows_ref, sem).wait()
                pltpu.sync_copy(rows_ref, dst.at[pl.ds(gbase + off, n)])
            else:
                pltpu.sync_copy(src.at[pl.ds(gbase + off, n)], rows_ref)
                pltpu.async_copy(rows_ref, dst.at[idx_ref], sem).wait()

        def unit_body(u, carry):
            unit = u * NW + wid

            @pl.when(unit < NU)
            def _():
                gbase = (unit // 2) * Lp + (unit % 2) * HALF

                def chunk_body(j, carry2):
                    do_chunk(gbase, j * SCCH, SCCH, idx_v, rows)
                    return carry2

                lax.fori_loop(0, NCH, chunk_body, 0)
                do_chunk(gbase, NCH * SCCH, TAIL, idx_t, rows_t)

            return carry

        lax.fori_loop(0, TRIPS, unit_body, 0)

    return functools.partial(
        pl.kernel, body, mesh=mesh,
        out_type=jax.ShapeDtypeStruct((G * Lp, ROW), jnp.float32),
        scratch_types=[
            pltpu.VMEM((SCCH,), jnp.int32),
            pltpu.VMEM((TAIL,), jnp.int32),
            pltpu.VMEM((SCCH, ROW), jnp.float32),
            pltpu.VMEM((TAIL, ROW), jnp.float32),
            pltpu.SemaphoreType.DMA,
        ])()


SORT_N = 16384
SORT_C = 128


def _bitonic_body(a_ref, idx_ref):
    Rr = SORT_N // SORT_C
    key = a_ref[0]
    r = lax.broadcasted_iota(jnp.int32, (Rr, SORT_C), 0)
    c = lax.broadcasted_iota(jnp.int32, (Rr, SORT_C), 1)
    ilin = r * SORT_C + c
    idx = ilin
    k = 2
    while k <= SORT_N:
        j = k // 2
        while j >= 1:
            if j >= SORT_C:
                s, axis, bit_src = j // SORT_C, 0, r
            else:
                s, axis, bit_src = j, 1, c
            bit = (bit_src & s) != 0
            pk = jnp.where(bit, jnp.roll(key, s, axis), jnp.roll(key, -s, axis))
            pi = jnp.where(bit, jnp.roll(idx, s, axis), jnp.roll(idx, -s, axis))
            lt = (key < pk) | ((key == pk) & (idx < pi))
            desc = (ilin & k) != 0
            take = jnp.logical_xor(jnp.logical_xor(lt, bit), desc)
            key = jnp.where(take, key, pk)
            idx = jnp.where(take, idx, pi)
            j //= 2
        k *= 2
    idx_ref[0] = idx


def _bitonic_argsort(angles2d):
    G, Lp = angles2d.shape
    pad = jnp.full((G, SORT_N - Lp), jnp.inf, jnp.float32)
    ap = jnp.concatenate([angles2d, pad], axis=1).reshape(G, SORT_N // SORT_C, SORT_C)
    idx = pl.pallas_call(
        _bitonic_body,
        grid=(G,),
        in_specs=[pl.BlockSpec((1, SORT_N // SORT_C, SORT_C), lambda g: (g, 0, 0))],
        out_specs=pl.BlockSpec((1, SORT_N // SORT_C, SORT_C), lambda g: (g, 0, 0)),
        out_shape=jax.ShapeDtypeStruct((G, SORT_N // SORT_C, SORT_C), jnp.int32),
    )(ap)
    return idx.reshape(G, SORT_N)[:, :Lp]


def kernel(x, W_qk, b_qk, W_v, b_v, W_o, b_o):
    Bn, Ln, D = x.shape
    H, dh, bs = N_HEADS, DH, BS
    pad_len = bs - (Ln % bs)
    xp = jnp.concatenate([x, jnp.zeros((Bn, pad_len, D), x.dtype)], axis=1)
    Lp = xp.shape[1]
    nt = Lp // LT
    GB = HG * Bn

    qk_all = xp @ W_qk.T + b_qk
    angles_l = []
    for h in range(H):
        qk = qk_all[:, :, h * dh:(h + 1) * dh]
        R = jax.random.normal(jax.random.fold_in(jax.random.key(42), h),
                              (dh, 2), dtype=jnp.float32)
        hout = lax.stop_gradient(qk) @ R
        angles_l.append(jnp.arctan(hout[:, :, 0] / hout[:, :, 1]))
    angles = jnp.stack(angles_l, axis=0)

    qkv_groups = pl.pallas_call(
        _qkvpack_body,
        grid=(Bn, nt),
        in_specs=[
            pl.BlockSpec((1, LT, D), lambda b, t: (b, t, 0)),
            pl.BlockSpec((1, LT, D), lambda b, t: (b, t, 0)),
            pl.BlockSpec((D, D), lambda b, t: (0, 0)),
            pl.BlockSpec((1, D), lambda b, t: (0, 0)),
        ],
        out_specs=[
            pl.BlockSpec((HG, 1, LT, ROW), lambda b, t: (0, b, t, 0))
        ] * NGRP,
        out_shape=[
            jax.ShapeDtypeStruct((HG, Bn, Lp, ROW), jnp.float32)
        ] * NGRP,
    )(xp, qk_all, W_v, b_v.reshape(1, D))

    offs = (jnp.arange(GB, dtype=jnp.int32) * Lp)[:, None]
    o_groups = []
    for g, qkv_g in enumerate(qkv_groups):
        ang_g = lax.dynamic_slice_in_dim(angles, g * HG, HG, axis=0)
        idx_g = _bitonic_argsort(ang_g.reshape(GB, Lp))
        idxp_g = (idx_g + offs).reshape(GB * Lp).astype(jnp.int32)
        qkv_s = _make_sc_permute(GB, Lp, reverse=False)(
            qkv_g.reshape(GB * Lp, ROW), idxp_g)
        o_s = pl.pallas_call(
            _attn_body,
            grid=(GB, Lp // LTA),
            in_specs=[pl.BlockSpec((1, LTA, ROW), lambda gg, t: (gg, t, 0))],
            out_specs=pl.BlockSpec((1, LTA, ROW), lambda gg, t: (gg, t, 0)),
            out_shape=jax.ShapeDtypeStruct((GB, Lp, ROW), jnp.float32),
        )(qkv_s.reshape(GB, Lp, ROW))
        o_g = _make_sc_permute(GB, Lp, reverse=True)(
            o_s.reshape(GB * Lp, ROW), idxp_g)
        o_groups.append(o_g.reshape(HG, Bn, Lp, ROW))

    out = pl.pallas_call(
        _outproj_body,
        grid=(Bn, nt),
        in_specs=[
            pl.BlockSpec((HG, 1, LT, ROW), lambda b, t: (0, b, t, 0))
        ] * NGRP + [
            pl.BlockSpec((D, D), lambda b, t: (0, 0)),
            pl.BlockSpec((1, D), lambda b, t: (0, 0)),
        ],
        out_specs=pl.BlockSpec((1, LT, D), lambda b, t: (b, t, 0)),
        out_shape=jax.ShapeDtypeStruct((Bn, Lp, D), jnp.float32),
    )(*o_groups, W_o, b_o.reshape(1, D))
    return out[:, :Ln]
